# baseline (speedup 1.0000x reference)
import jax
import jax.numpy as jnp
from jax import lax
from jax.experimental import pallas as pl
from jax.experimental.pallas import tpu as pltpu
from jax.experimental.pallas import tpu_sc as plsc

N = 10000
D = 128
E = 320000

NC = 2
NS = 16
NW = NC * NS

NPAD = 10240
EPW = 10240
EPAD = NW * EPW
DEG_CH = EPW // 16
RPT = NPAD // NS

CHUNK = 64
NCHUNKS = EPAD // CHUNK
SPT = NCHUNKS // NS

BLK = 512
NBLK = NPAD // BLK


def _deg_body(didx_hbm, out_hbm, didx_v, deg_v):
    cid = lax.axis_index("c")
    sid = lax.axis_index("s")
    wid = cid * NS + sid
    pltpu.sync_copy(didx_hbm.at[wid], didx_v)

    def zero_body(i, carry):
        deg_v[pl.ds(i * 16, 16)] = jnp.zeros((16,), jnp.float32)
        return carry

    lax.fori_loop(0, NPAD // 16, zero_body, 0)

    ones = jnp.ones((16,), jnp.float32)

    def acc_body(c, carry):
        idx = didx_v[c]
        plsc.addupdate_scatter(deg_v, [idx], ones)
        return carry

    lax.fori_loop(0, DEG_CH, acc_body, 0)
    pltpu.sync_copy(deg_v, out_hbm.at[wid])


def _deg_pass(dst16):
    return pl.kernel(
        _deg_body,
        out_type=jax.ShapeDtypeStruct((NW, NPAD), jnp.float32),
        mesh=plsc.VectorSubcoreMesh(core_axis_name="c", subcore_axis_name="s"),
        compiler_params=pltpu.CompilerParams(needs_layout_passes=False),
        scratch_types=[
            pltpu.VMEM((DEG_CH, 16), jnp.int32),
            pltpu.VMEM((NPAD,), jnp.float32),
        ],
    )(dst16)


def _copy_idx(pk_v, sidx_v, didx_v, slot):
    for w in range(CHUNK // 16):
        sidx_v[slot, pl.ds(w * 16, 16)] = pk_v[slot, 0, pl.ds(w * 16, 16)]
        didx_v[slot, pl.ds(w * 16, 16)] = pk_v[slot, 1, pl.ds(w * 16, 16)]


def _spmm_body(table_hbm, pk_hbm, zeros_hbm, out_hbm,
               pk_v, sidx_v, didx_v, rows_v, acc_sh, *sems):
    semr = sems[0:4]
    semw = sems[4:8]
    semi = sems[8:12]
    cid = lax.axis_index("c")
    sid = lax.axis_index("s")
    r0 = sid * RPT

    @pl.when(cid == 0)
    def _():
        base = sid * SPT
        pltpu.sync_copy(zeros_hbm.at[pl.ds(r0, RPT)],
                        acc_sh.at[pl.ds(r0, RPT)])
        plsc.subcore_barrier()

        for k in range(2):
            pltpu.sync_copy(pk_hbm.at[base + k], pk_v.at[k])
            _copy_idx(pk_v, sidx_v, didx_v, k)
            pltpu.async_copy(
                table_hbm.at[sidx_v.at[k]], rows_v.at[k], semr[k])
        for j in range(2, 6):
            pltpu.async_copy(pk_hbm.at[base + j], pk_v.at[j % 4],
                             semi[j % 4])

        def quad_body(c4, carry):
            for k in range(4):
                c = 4 * c4 + k
                b = k
                b2 = (k + 2) % 4
                pltpu.make_async_copy(
                    table_hbm.at[sidx_v.at[b]], rows_v.at[b],
                    semr[b]).wait()
                pltpu.async_copy(
                    rows_v.at[b], acc_sh.at[didx_v.at[b]], semw[b],
                    add=True)

                def _drain():
                    pltpu.make_async_copy(
                        rows_v.at[b2], acc_sh.at[didx_v.at[b2]],
                        semw[b2]).wait()

                if k >= 2:
                    _drain()
                else:
                    @pl.when(c4 > 0)
                    def _():
                        _drain()

                @pl.when(c + 2 < SPT)
                def _():
                    pltpu.make_async_copy(
                        pk_hbm.at[base + c + 2], pk_v.at[b2],
                        semi[b2]).wait()
                    _copy_idx(pk_v, sidx_v, didx_v, b2)
                    pltpu.async_copy(
                        table_hbm.at[sidx_v.at[b2]], rows_v.at[b2],
                        semr[b2])

                @pl.when(c + 6 < SPT)
                def _():
                    pltpu.async_copy(pk_hbm.at[base + c + 6], pk_v.at[b2],
                                     semi[b2])
            return carry

        lax.fori_loop(0, SPT // 4, quad_body, 0)
        for b in (2, 3):
            pltpu.make_async_copy(
                rows_v.at[b], acc_sh.at[didx_v.at[b]], semw[b]).wait()
        plsc.subcore_barrier()
        pltpu.sync_copy(acc_sh.at[pl.ds(r0, RPT)],
                        out_hbm.at[pl.ds(r0, RPT)])


def _spmm_pass(table, pkr, zeros_tab):
    return pl.kernel(
        _spmm_body,
        out_type=jax.ShapeDtypeStruct((NPAD, D), jnp.float32),
        mesh=plsc.VectorSubcoreMesh(core_axis_name="c", subcore_axis_name="s"),
        compiler_params=pltpu.CompilerParams(needs_layout_passes=False),
        scratch_types=[
            pltpu.VMEM((4, 2, CHUNK), jnp.int32),
            pltpu.VMEM((4, CHUNK), jnp.int32),
            pltpu.VMEM((4, CHUNK), jnp.int32),
            pltpu.VMEM((4, CHUNK, D), jnp.float32),
            pltpu.VMEM_SHARED((NPAD, D), jnp.float32),
        ] + [pltpu.SemaphoreType.DMA] * 12,
    )(table, pkr, zeros_tab)


def _dec_body(z_hbm, pk_hbm, out_hbm,
              pk_v, sidx_v, didx_v, za_v, zb_v, p_v, sc_v, *sems):
    semd = sems[0:4]
    semw = sems[4:8]
    semi = sems[8:12]
    cid = lax.axis_index("c")
    sid = lax.axis_index("s")
    lane = lax.broadcasted_iota(jnp.int32, (16,), 0)

    @pl.when(cid == 0)
    def _():
        base = sid * SPT

        for k in range(4):
            pltpu.sync_copy(pk_hbm.at[base + k], pk_v.at[k])
            _copy_idx(pk_v, sidx_v, didx_v, k)
            pltpu.async_copy(z_hbm.at[sidx_v.at[k]], za_v.at[k], semd[k])
            pltpu.async_copy(z_hbm.at[didx_v.at[k]], zb_v.at[k], semd[k])
        for j in range(4, 8):
            pltpu.async_copy(pk_hbm.at[base + j], pk_v.at[j % 4],
                             semi[j % 4])

        def quad_body(c4, carry):
            for k in range(4):
                c = 4 * c4 + k
                b = k
                pltpu.make_async_copy(
                    z_hbm.at[sidx_v.at[b]], za_v.at[b], semd[b]).wait()
                pltpu.make_async_copy(
                    z_hbm.at[didx_v.at[b]], zb_v.at[b], semd[b]).wait()

                @pl.when(c4 > 0)
                def _():
                    pltpu.make_async_copy(
                        sc_v.at[b], out_hbm.at[pl.ds(0, CHUNK)],
                        semw[b]).wait()

                def group_body(g, carry2):
                    def edge_body(i, carry3):
                        e = g * 16 + i
                        a = (za_v[b, e, pl.ds(0, 16)]
                             * zb_v[b, e, pl.ds(0, 16)])
                        for j in range(1, D // 16):
                            a = a + (za_v[b, e, pl.ds(j * 16, 16)]
                                     * zb_v[b, e, pl.ds(j * 16, 16)])
                        p_v[i, pl.ds(0, 16)] = a
                        return carry3

                    lax.fori_loop(0, 16, edge_body, 0)
                    acc = plsc.load_gather(
                        p_v, [lane, jnp.zeros((16,), jnp.int32)])
                    for i in range(1, 16):
                        acc = acc + plsc.load_gather(
                            p_v, [lane, jnp.full((16,), i, jnp.int32)])
                    sc_v[b, pl.ds(g * 16, 16)] = acc
                    return carry2

                lax.fori_loop(0, CHUNK // 16, group_body, 0)
                pltpu.async_copy(
                    sc_v.at[b],
                    out_hbm.at[pl.ds((base + c) * CHUNK, CHUNK)], semw[b])

                @pl.when(c + 4 < SPT)
                def _():
                    pltpu.make_async_copy(
                        pk_hbm.at[base + c + 4], pk_v.at[b], semi[b]).wait()
                    _copy_idx(pk_v, sidx_v, didx_v, b)
                    pltpu.async_copy(
                        z_hbm.at[sidx_v.at[b]], za_v.at[b], semd[b])
                    pltpu.async_copy(
                        z_hbm.at[didx_v.at[b]], zb_v.at[b], semd[b])

                @pl.when(c + 8 < SPT)
                def _():
                    pltpu.async_copy(pk_hbm.at[base + c + 8], pk_v.at[b],
                                     semi[b])
            return carry

        lax.fori_loop(0, SPT // 4, quad_body, 0)
        for b in range(4):
            pltpu.make_async_copy(
                sc_v.at[b], out_hbm.at[pl.ds(0, CHUNK)], semw[b]).wait()


def _dec_pass(z, pkr):
    return pl.kernel(
        _dec_body,
        out_type=jax.ShapeDtypeStruct((EPAD,), jnp.float32),
        mesh=plsc.VectorSubcoreMesh(core_axis_name="c", subcore_axis_name="s"),
        compiler_params=pltpu.CompilerParams(needs_layout_passes=False),
        scratch_types=[
            pltpu.VMEM((4, 2, CHUNK), jnp.int32),
            pltpu.VMEM((4, CHUNK), jnp.int32),
            pltpu.VMEM((4, CHUNK), jnp.int32),
            pltpu.VMEM((4, CHUNK, D), jnp.float32),
            pltpu.VMEM((4, CHUNK, D), jnp.float32),
            pltpu.VMEM((16, 17), jnp.float32),
            pltpu.VMEM((4, CHUNK), jnp.float32),
        ] + [pltpu.SemaphoreType.DMA] * 12,
    )(z, pkr)


def _dis_block(dp_block, block_id):
    deg = jnp.sum(dp_block, axis=0)
    rows = block_id * BLK + lax.broadcasted_iota(jnp.int32, (BLK,), 0)
    real = rows < N
    deg = deg + real.astype(jnp.float32)
    dis = jnp.where(real, lax.rsqrt(deg), 0.0)
    return dis[:, None]


def _tc1_body(x_ref, w_ref, dp_ref, o_ref):
    i = pl.program_id(0)
    dis = _dis_block(dp_ref[...], i)
    xw = jnp.dot(x_ref[...], w_ref[...], preferred_element_type=jnp.float32)
    o_ref[...] = xw * dis


def _tc1(xpad, W1, deg_part):
    return pl.pallas_call(
        _tc1_body,
        out_shape=jax.ShapeDtypeStruct((NPAD, D), jnp.float32),
        grid=(NBLK,),
        in_specs=[
            pl.BlockSpec((BLK, D), lambda i: (i, 0)),
            pl.BlockSpec((D, D), lambda i: (0, 0)),
            pl.BlockSpec((NW, BLK), lambda i: (0, i)),
        ],
        out_specs=pl.BlockSpec((BLK, D), lambda i: (i, 0)),
    )(xpad, W1, deg_part)


def _tc2_body(acc_ref, xws_ref, w_ref, b_ref, dp_ref, o_ref):
    i = pl.program_id(0)
    dis = _dis_block(dp_ref[...], i)
    s = (acc_ref[...] + xws_ref[...]) * dis
    h = jnp.maximum(s + b_ref[...], 0.0)
    o_ref[...] = jnp.dot(h, w_ref[...], preferred_element_type=jnp.float32) * dis


def _tc2(acc, xws1, W2, b1r, deg_part):
    return pl.pallas_call(
        _tc2_body,
        out_shape=jax.ShapeDtypeStruct((NPAD, D), jnp.float32),
        grid=(NBLK,),
        in_specs=[
            pl.BlockSpec((BLK, D), lambda i: (i, 0)),
            pl.BlockSpec((BLK, D), lambda i: (i, 0)),
            pl.BlockSpec((D, D), lambda i: (0, 0)),
            pl.BlockSpec((1, D), lambda i: (0, 0)),
            pl.BlockSpec((NW, BLK), lambda i: (0, i)),
        ],
        out_specs=pl.BlockSpec((BLK, D), lambda i: (i, 0)),
    )(acc, xws1, W2, b1r, deg_part)


def _tc3_body(acc_ref, xws_ref, b_ref, dp_ref, o_ref):
    i = pl.program_id(0)
    dis = _dis_block(dp_ref[...], i)
    o_ref[...] = (acc_ref[...] + xws_ref[...]) * dis + b_ref[...]


def _tc3(acc, xws2, b2r, deg_part):
    return pl.pallas_call(
        _tc3_body,
        out_shape=jax.ShapeDtypeStruct((NPAD, D), jnp.float32),
        grid=(NBLK,),
        in_specs=[
            pl.BlockSpec((BLK, D), lambda i: (i, 0)),
            pl.BlockSpec((BLK, D), lambda i: (i, 0)),
            pl.BlockSpec((1, D), lambda i: (0, 0)),
            pl.BlockSpec((NW, BLK), lambda i: (0, i)),
        ],
        out_specs=pl.BlockSpec((BLK, D), lambda i: (i, 0)),
    )(acc, xws2, b2r, deg_part)


def kernel(x, edge_index, W1, b1, W2, b2):
    ei = edge_index.astype(jnp.int32)
    src = ei[0]
    dst = ei[1]
    pad = jnp.full((EPAD - E,), N, jnp.int32)
    srcp = jnp.concatenate([src, pad])
    dstp = jnp.concatenate([dst, pad])
    pkr = jnp.stack(
        [srcp.reshape(NCHUNKS, CHUNK), dstp.reshape(NCHUNKS, CHUNK)], axis=1)
    dst16 = dstp.reshape(NW, DEG_CH, 16)

    xpad = jnp.concatenate(
        [x.astype(jnp.float32), jnp.zeros((NPAD - N, D), jnp.float32)])
    zeros_tab = jnp.zeros((NPAD, D), jnp.float32)
    b1r = b1.reshape(1, D).astype(jnp.float32)
    b2r = b2.reshape(1, D).astype(jnp.float32)

    deg_part = _deg_pass(dst16)
    xws1 = _tc1(xpad, W1, deg_part)
    acc1 = _spmm_pass(xws1, pkr, zeros_tab)
    xws2 = _tc2(acc1, xws1, W2, b1r, deg_part)
    acc2 = _spmm_pass(xws2, pkr, zeros_tab)
    z = _tc3(acc2, xws2, b2r, deg_part)
    scores_pad = _dec_pass(z, pkr)
    return scores_pad[:E]


# speedup vs baseline: 1.0637x; 1.0637x over previous
import jax
import jax.numpy as jnp
from jax import lax
from jax.experimental import pallas as pl
from jax.experimental.pallas import tpu as pltpu
from jax.experimental.pallas import tpu_sc as plsc

N = 10000
D = 128
E = 320000

NC = 2
NS = 16
NW = NC * NS

NPAD = 10240
EPW = 10240
EPAD = NW * EPW
DEG_CH = EPW // 16
RPT = NPAD // NS

CHUNK = 64
NCHUNKS = EPAD // CHUNK
SPT = NCHUNKS // NS
SN0 = 256
SN1 = 64

BLK = 512
NBLK = NPAD // BLK


def _deg_body(didx_hbm, out_hbm, didx_v, deg_v):
    cid = lax.axis_index("c")
    sid = lax.axis_index("s")
    wid = cid * NS + sid
    pltpu.sync_copy(didx_hbm.at[wid], didx_v)

    def zero_body(i, carry):
        deg_v[pl.ds(i * 16, 16)] = jnp.zeros((16,), jnp.float32)
        return carry

    lax.fori_loop(0, NPAD // 16, zero_body, 0)

    ones = jnp.ones((16,), jnp.float32)

    def acc_body(c, carry):
        idx = didx_v[c]
        plsc.addupdate_scatter(deg_v, [idx], ones)
        return carry

    lax.fori_loop(0, DEG_CH, acc_body, 0)
    pltpu.sync_copy(deg_v, out_hbm.at[wid])


def _deg_pass(dst16):
    return pl.kernel(
        _deg_body,
        out_type=jax.ShapeDtypeStruct((NW, NPAD), jnp.float32),
        mesh=plsc.VectorSubcoreMesh(core_axis_name="c", subcore_axis_name="s"),
        compiler_params=pltpu.CompilerParams(needs_layout_passes=False),
        scratch_types=[
            pltpu.VMEM((DEG_CH, 16), jnp.int32),
            pltpu.VMEM((NPAD,), jnp.float32),
        ],
    )(dst16)


def _copy_idx(pk_v, sidx_v, didx_v, slot):
    for w in range(CHUNK // 16):
        sidx_v[slot, pl.ds(w * 16, 16)] = pk_v[slot, 0, pl.ds(w * 16, 16)]
        didx_v[slot, pl.ds(w * 16, 16)] = pk_v[slot, 1, pl.ds(w * 16, 16)]


def _spmm_body(table_hbm, pk_hbm, zeros_hbm, out_hbm,
               pk_v, sidx_v, didx_v, rows_v, acc_sh, *sems):
    semr = sems[0:4]
    semw = sems[4:8]
    semi = sems[8:12]
    cid = lax.axis_index("c")
    sid = lax.axis_index("s")
    r0 = sid * RPT

    if True:
        base = jnp.where(cid == 0, sid * SN0, NS * SN0 + sid * SN1)
        count = jnp.where(cid == 0, SN0, SN1)
        pltpu.sync_copy(zeros_hbm.at[pl.ds(r0, RPT)],
                        acc_sh.at[pl.ds(r0, RPT)])
        plsc.subcore_barrier()

        for k in range(2):
            pltpu.sync_copy(pk_hbm.at[base + k], pk_v.at[k])
            _copy_idx(pk_v, sidx_v, didx_v, k)
            pltpu.async_copy(
                table_hbm.at[sidx_v.at[k]], rows_v.at[k], semr[k])
        for j in range(2, 6):
            pltpu.async_copy(pk_hbm.at[base + j], pk_v.at[j % 4],
                             semi[j % 4])

        def quad_body(c4, carry):
            for k in range(4):
                c = 4 * c4 + k
                b = k
                b2 = (k + 2) % 4
                pltpu.make_async_copy(
                    table_hbm.at[sidx_v.at[b]], rows_v.at[b],
                    semr[b]).wait()
                pltpu.async_copy(
                    rows_v.at[b], acc_sh.at[didx_v.at[b]], semw[b],
                    add=True)

                def _drain():
                    pltpu.make_async_copy(
                        rows_v.at[b2], acc_sh.at[didx_v.at[b2]],
                        semw[b2]).wait()

                if k >= 2:
                    _drain()
                else:
                    @pl.when(c4 > 0)
                    def _():
                        _drain()

                @pl.when(c + 2 < count)
                def _():
                    pltpu.make_async_copy(
                        pk_hbm.at[base + c + 2], pk_v.at[b2],
                        semi[b2]).wait()
                    _copy_idx(pk_v, sidx_v, didx_v, b2)
                    pltpu.async_copy(
                        table_hbm.at[sidx_v.at[b2]], rows_v.at[b2],
                        semr[b2])

                @pl.when(c + 6 < count)
                def _():
                    pltpu.async_copy(pk_hbm.at[base + c + 6], pk_v.at[b2],
                                     semi[b2])
            return carry

        lax.fori_loop(0, count // 4, quad_body, 0)
        for b in (2, 3):
            pltpu.make_async_copy(
                rows_v.at[b], acc_sh.at[didx_v.at[b]], semw[b]).wait()
        plsc.subcore_barrier()
        pltpu.sync_copy(acc_sh.at[pl.ds(r0, RPT)],
                        out_hbm.at[cid, pl.ds(r0, RPT)])


def _spmm_pass(table, pkr, zeros_tab):
    return pl.kernel(
        _spmm_body,
        out_type=jax.ShapeDtypeStruct((NC, NPAD, D), jnp.float32),
        mesh=plsc.VectorSubcoreMesh(core_axis_name="c", subcore_axis_name="s"),
        compiler_params=pltpu.CompilerParams(needs_layout_passes=False),
        scratch_types=[
            pltpu.VMEM((4, 2, CHUNK), jnp.int32),
            pltpu.VMEM((4, CHUNK), jnp.int32),
            pltpu.VMEM((4, CHUNK), jnp.int32),
            pltpu.VMEM((4, CHUNK, D), jnp.float32),
            pltpu.VMEM_SHARED((NPAD, D), jnp.float32),
        ] + [pltpu.SemaphoreType.DMA] * 12,
    )(table, pkr, zeros_tab)


def _dec_body(z_hbm, pk_hbm, out_hbm,
              pk_v, sidx_v, didx_v, za_v, zb_v, p_v, sc_v, *sems):
    semd = sems[0:4]
    semw = sems[4:8]
    semi = sems[8:12]
    cid = lax.axis_index("c")
    sid = lax.axis_index("s")
    lane = lax.broadcasted_iota(jnp.int32, (16,), 0)

    @pl.when(cid == 0)
    def _():
        base = sid * SPT

        for k in range(4):
            pltpu.sync_copy(pk_hbm.at[base + k], pk_v.at[k])
            _copy_idx(pk_v, sidx_v, didx_v, k)
            pltpu.async_copy(z_hbm.at[sidx_v.at[k]], za_v.at[k], semd[k])
            pltpu.async_copy(z_hbm.at[didx_v.at[k]], zb_v.at[k], semd[k])
        for j in range(4, 8):
            pltpu.async_copy(pk_hbm.at[base + j], pk_v.at[j % 4],
                             semi[j % 4])

        def quad_body(c4, carry):
            for k in range(4):
                c = 4 * c4 + k
                b = k
                pltpu.make_async_copy(
                    z_hbm.at[sidx_v.at[b]], za_v.at[b], semd[b]).wait()
                pltpu.make_async_copy(
                    z_hbm.at[didx_v.at[b]], zb_v.at[b], semd[b]).wait()

                @pl.when(c4 > 0)
                def _():
                    pltpu.make_async_copy(
                        sc_v.at[b], out_hbm.at[pl.ds(0, CHUNK)],
                        semw[b]).wait()

                def group_body(g, carry2):
                    def edge_body(i, carry3):
                        e = g * 16 + i
                        a = (za_v[b, e, pl.ds(0, 16)]
                             * zb_v[b, e, pl.ds(0, 16)])
                        for j in range(1, D // 16):
                            a = a + (za_v[b, e, pl.ds(j * 16, 16)]
                                     * zb_v[b, e, pl.ds(j * 16, 16)])
                        p_v[i, pl.ds(0, 16)] = a
                        return carry3

                    lax.fori_loop(0, 16, edge_body, 0)
                    acc = plsc.load_gather(
                        p_v, [lane, jnp.zeros((16,), jnp.int32)])
                    for i in range(1, 16):
                        acc = acc + plsc.load_gather(
                            p_v, [lane, jnp.full((16,), i, jnp.int32)])
                    sc_v[b, pl.ds(g * 16, 16)] = acc
                    return carry2

                lax.fori_loop(0, CHUNK // 16, group_body, 0)
                pltpu.async_copy(
                    sc_v.at[b],
                    out_hbm.at[pl.ds((base + c) * CHUNK, CHUNK)], semw[b])

                @pl.when(c + 4 < SPT)
                def _():
                    pltpu.make_async_copy(
                        pk_hbm.at[base + c + 4], pk_v.at[b], semi[b]).wait()
                    _copy_idx(pk_v, sidx_v, didx_v, b)
                    pltpu.async_copy(
                        z_hbm.at[sidx_v.at[b]], za_v.at[b], semd[b])
                    pltpu.async_copy(
                        z_hbm.at[didx_v.at[b]], zb_v.at[b], semd[b])

                @pl.when(c + 8 < SPT)
                def _():
                    pltpu.async_copy(pk_hbm.at[base + c + 8], pk_v.at[b],
                                     semi[b])
            return carry

        lax.fori_loop(0, SPT // 4, quad_body, 0)
        for b in range(4):
            pltpu.make_async_copy(
                sc_v.at[b], out_hbm.at[pl.ds(0, CHUNK)], semw[b]).wait()


def _dec_pass(z, pkr):
    return pl.kernel(
        _dec_body,
        out_type=jax.ShapeDtypeStruct((EPAD,), jnp.float32),
        mesh=plsc.VectorSubcoreMesh(core_axis_name="c", subcore_axis_name="s"),
        compiler_params=pltpu.CompilerParams(needs_layout_passes=False),
        scratch_types=[
            pltpu.VMEM((4, 2, CHUNK), jnp.int32),
            pltpu.VMEM((4, CHUNK), jnp.int32),
            pltpu.VMEM((4, CHUNK), jnp.int32),
            pltpu.VMEM((4, CHUNK, D), jnp.float32),
            pltpu.VMEM((4, CHUNK, D), jnp.float32),
            pltpu.VMEM((16, 17), jnp.float32),
            pltpu.VMEM((4, CHUNK), jnp.float32),
        ] + [pltpu.SemaphoreType.DMA] * 12,
    )(z, pkr)


def _dis_block(dp_block, block_id):
    deg = jnp.sum(dp_block, axis=0)
    rows = block_id * BLK + lax.broadcasted_iota(jnp.int32, (BLK,), 0)
    real = rows < N
    deg = deg + real.astype(jnp.float32)
    dis = jnp.where(real, lax.rsqrt(deg), 0.0)
    return dis[:, None]


def _tc1_body(x_ref, w_ref, dp_ref, o_ref):
    i = pl.program_id(0)
    dis = _dis_block(dp_ref[...], i)
    xw = jnp.dot(x_ref[...], w_ref[...], preferred_element_type=jnp.float32)
    o_ref[...] = xw * dis


def _tc1(xpad, W1, deg_part):
    return pl.pallas_call(
        _tc1_body,
        out_shape=jax.ShapeDtypeStruct((NPAD, D), jnp.float32),
        grid=(NBLK,),
        in_specs=[
            pl.BlockSpec((BLK, D), lambda i: (i, 0)),
            pl.BlockSpec((D, D), lambda i: (0, 0)),
            pl.BlockSpec((NW, BLK), lambda i: (0, i)),
        ],
        out_specs=pl.BlockSpec((BLK, D), lambda i: (i, 0)),
    )(xpad, W1, deg_part)


def _tc2_body(acc_ref, xws_ref, w_ref, b_ref, dp_ref, o_ref):
    i = pl.program_id(0)
    dis = _dis_block(dp_ref[...], i)
    s = (acc_ref[0] + acc_ref[1] + xws_ref[...]) * dis
    h = jnp.maximum(s + b_ref[...], 0.0)
    o_ref[...] = jnp.dot(h, w_ref[...], preferred_element_type=jnp.float32) * dis


def _tc2(acc, xws1, W2, b1r, deg_part):
    return pl.pallas_call(
        _tc2_body,
        out_shape=jax.ShapeDtypeStruct((NPAD, D), jnp.float32),
        grid=(NBLK,),
        in_specs=[
            pl.BlockSpec((NC, BLK, D), lambda i: (0, i, 0)),
            pl.BlockSpec((BLK, D), lambda i: (i, 0)),
            pl.BlockSpec((D, D), lambda i: (0, 0)),
            pl.BlockSpec((1, D), lambda i: (0, 0)),
            pl.BlockSpec((NW, BLK), lambda i: (0, i)),
        ],
        out_specs=pl.BlockSpec((BLK, D), lambda i: (i, 0)),
    )(acc, xws1, W2, b1r, deg_part)


def _tc3_body(acc_ref, xws_ref, b_ref, dp_ref, o_ref):
    i = pl.program_id(0)
    dis = _dis_block(dp_ref[...], i)
    o_ref[...] = (acc_ref[0] + acc_ref[1] + xws_ref[...]) * dis + b_ref[...]


def _tc3(acc, xws2, b2r, deg_part):
    return pl.pallas_call(
        _tc3_body,
        out_shape=jax.ShapeDtypeStruct((NPAD, D), jnp.float32),
        grid=(NBLK,),
        in_specs=[
            pl.BlockSpec((NC, BLK, D), lambda i: (0, i, 0)),
            pl.BlockSpec((BLK, D), lambda i: (i, 0)),
            pl.BlockSpec((1, D), lambda i: (0, 0)),
            pl.BlockSpec((NW, BLK), lambda i: (0, i)),
        ],
        out_specs=pl.BlockSpec((BLK, D), lambda i: (i, 0)),
    )(acc, xws2, b2r, deg_part)


def kernel(x, edge_index, W1, b1, W2, b2):
    ei = edge_index.astype(jnp.int32)
    src = ei[0]
    dst = ei[1]
    pad = jnp.full((EPAD - E,), N, jnp.int32)
    srcp = jnp.concatenate([src, pad])
    dstp = jnp.concatenate([dst, pad])
    pkr = jnp.stack(
        [srcp.reshape(NCHUNKS, CHUNK), dstp.reshape(NCHUNKS, CHUNK)], axis=1)
    dst16 = dstp.reshape(NW, DEG_CH, 16)

    xpad = jnp.concatenate(
        [x.astype(jnp.float32), jnp.zeros((NPAD - N, D), jnp.float32)])
    zeros_tab = jnp.zeros((NPAD, D), jnp.float32)
    b1r = b1.reshape(1, D).astype(jnp.float32)
    b2r = b2.reshape(1, D).astype(jnp.float32)

    deg_part = _deg_pass(dst16)
    xws1 = _tc1(xpad, W1, deg_part)
    acc1 = _spmm_pass(xws1, pkr, zeros_tab)
    xws2 = _tc2(acc1, xws1, W2, b1r, deg_part)
    acc2 = _spmm_pass(xws2, pkr, zeros_tab)
    z = _tc3(acc2, xws2, b2r, deg_part)
    scores_pad = _dec_pass(z, pkr)
    return scores_pad[:E]


# speedup vs baseline: 1.3193x; 1.2403x over previous
import jax
import jax.numpy as jnp
from jax import lax
from jax.experimental import pallas as pl
from jax.experimental.pallas import tpu as pltpu
from jax.experimental.pallas import tpu_sc as plsc

N = 10000
D = 128
E = 320000

NC = 2
NS = 16
NW = NC * NS

NPAD = 10240
EPW = 10240
EPAD = NW * EPW
DEG_CH = EPW // 16
RPT = NPAD // NS

CHUNK = 64
NCHUNKS = EPAD // CHUNK
SPT = NCHUNKS // NS
SN0 = 256
SN1 = 64
DN0 = 288
DN1 = 32

BLK = 512
NBLK = NPAD // BLK


def _deg_body(didx_hbm, out_hbm, didx_v, deg_v):
    cid = lax.axis_index("c")
    sid = lax.axis_index("s")
    wid = cid * NS + sid
    pltpu.sync_copy(didx_hbm.at[wid], didx_v)

    def zero_body(i, carry):
        deg_v[pl.ds(i * 16, 16)] = jnp.zeros((16,), jnp.float32)
        return carry

    lax.fori_loop(0, NPAD // 16, zero_body, 0)

    ones = jnp.ones((16,), jnp.float32)

    def acc_body(c, carry):
        idx = didx_v[c]
        plsc.addupdate_scatter(deg_v, [idx], ones)
        return carry

    lax.fori_loop(0, DEG_CH, acc_body, 0)
    pltpu.sync_copy(deg_v, out_hbm.at[wid])


def _deg_pass(dst16):
    return pl.kernel(
        _deg_body,
        out_type=jax.ShapeDtypeStruct((NW, NPAD), jnp.float32),
        mesh=plsc.VectorSubcoreMesh(core_axis_name="c", subcore_axis_name="s"),
        compiler_params=pltpu.CompilerParams(needs_layout_passes=False),
        scratch_types=[
            pltpu.VMEM((DEG_CH, 16), jnp.int32),
            pltpu.VMEM((NPAD,), jnp.float32),
        ],
    )(dst16)


def _copy_idx(pk_v, sidx_v, didx_v, slot):
    for w in range(CHUNK // 16):
        sidx_v[slot, pl.ds(w * 16, 16)] = pk_v[slot, 0, pl.ds(w * 16, 16)]
        didx_v[slot, pl.ds(w * 16, 16)] = pk_v[slot, 1, pl.ds(w * 16, 16)]


def _spmm_body(table_hbm, pk_hbm, zeros_hbm, out_hbm,
               pk_v, sidx_v, didx_v, rows_v, acc_sh, *sems):
    semr = sems[0:4]
    semw = sems[4:8]
    semi = sems[8:12]
    cid = lax.axis_index("c")
    sid = lax.axis_index("s")
    r0 = sid * RPT

    if True:
        base = jnp.where(cid == 0, sid * SN0, NS * SN0 + sid * SN1)
        count = jnp.where(cid == 0, SN0, SN1)
        pltpu.sync_copy(zeros_hbm.at[pl.ds(r0, RPT)],
                        acc_sh.at[pl.ds(r0, RPT)])
        plsc.subcore_barrier()

        for k in range(2):
            pltpu.sync_copy(pk_hbm.at[base + k], pk_v.at[k])
            _copy_idx(pk_v, sidx_v, didx_v, k)
            pltpu.async_copy(
                table_hbm.at[sidx_v.at[k]], rows_v.at[k], semr[k])
        for j in range(2, 6):
            pltpu.async_copy(pk_hbm.at[base + j], pk_v.at[j % 4],
                             semi[j % 4])

        def quad_body(c4, carry):
            for k in range(4):
                c = 4 * c4 + k
                b = k
                b2 = (k + 2) % 4
                pltpu.make_async_copy(
                    table_hbm.at[sidx_v.at[b]], rows_v.at[b],
                    semr[b]).wait()
                pltpu.async_copy(
                    rows_v.at[b], acc_sh.at[didx_v.at[b]], semw[b],
                    add=True)

                def _drain():
                    pltpu.make_async_copy(
                        rows_v.at[b2], acc_sh.at[didx_v.at[b2]],
                        semw[b2]).wait()

                if k >= 2:
                    _drain()
                else:
                    @pl.when(c4 > 0)
                    def _():
                        _drain()

                @pl.when(c + 2 < count)
                def _():
                    pltpu.make_async_copy(
                        pk_hbm.at[base + c + 2], pk_v.at[b2],
                        semi[b2]).wait()
                    _copy_idx(pk_v, sidx_v, didx_v, b2)
                    pltpu.async_copy(
                        table_hbm.at[sidx_v.at[b2]], rows_v.at[b2],
                        semr[b2])

                @pl.when(c + 6 < count)
                def _():
                    pltpu.async_copy(pk_hbm.at[base + c + 6], pk_v.at[b2],
                                     semi[b2])
            return carry

        lax.fori_loop(0, count // 4, quad_body, 0)
        for b in (2, 3):
            pltpu.make_async_copy(
                rows_v.at[b], acc_sh.at[didx_v.at[b]], semw[b]).wait()
        plsc.subcore_barrier()
        pltpu.sync_copy(acc_sh.at[pl.ds(r0, RPT)],
                        out_hbm.at[cid, pl.ds(r0, RPT)])


def _spmm_pass(table, pkr, zeros_tab):
    return pl.kernel(
        _spmm_body,
        out_type=jax.ShapeDtypeStruct((NC, NPAD, D), jnp.float32),
        mesh=plsc.VectorSubcoreMesh(core_axis_name="c", subcore_axis_name="s"),
        compiler_params=pltpu.CompilerParams(needs_layout_passes=False),
        scratch_types=[
            pltpu.VMEM((4, 2, CHUNK), jnp.int32),
            pltpu.VMEM((4, CHUNK), jnp.int32),
            pltpu.VMEM((4, CHUNK), jnp.int32),
            pltpu.VMEM((4, CHUNK, D), jnp.float32),
            pltpu.VMEM_SHARED((NPAD, D), jnp.float32),
        ] + [pltpu.SemaphoreType.DMA] * 12,
    )(table, pkr, zeros_tab)


def _dec_body(z_hbm, pk_hbm, out_hbm,
              pk_v, sidx_v, didx_v, za_v, zb_v, p_v, sc_v, *sems):
    semd = sems[0:4]
    semw = sems[4:8]
    semi = sems[8:12]
    cid = lax.axis_index("c")
    sid = lax.axis_index("s")
    lane = lax.broadcasted_iota(jnp.int32, (16,), 0)

    if True:
        base = jnp.where(cid == 0, sid * DN0, NS * DN0 + sid * DN1)
        count = jnp.where(cid == 0, DN0, DN1)

        for k in range(4):
            pltpu.sync_copy(pk_hbm.at[base + k], pk_v.at[k])
            _copy_idx(pk_v, sidx_v, didx_v, k)
            pltpu.async_copy(z_hbm.at[sidx_v.at[k]], za_v.at[k], semd[k])
            pltpu.async_copy(z_hbm.at[didx_v.at[k]], zb_v.at[k], semd[k])
        for j in range(4, 8):
            pltpu.async_copy(pk_hbm.at[base + j], pk_v.at[j % 4],
                             semi[j % 4])

        def quad_body(c4, carry):
            for k in range(4):
                c = 4 * c4 + k
                b = k
                pltpu.make_async_copy(
                    z_hbm.at[sidx_v.at[b]], za_v.at[b], semd[b]).wait()
                pltpu.make_async_copy(
                    z_hbm.at[didx_v.at[b]], zb_v.at[b], semd[b]).wait()

                @pl.when(c4 > 0)
                def _():
                    pltpu.make_async_copy(
                        sc_v.at[b], out_hbm.at[pl.ds(0, CHUNK)],
                        semw[b]).wait()

                def group_body(g, carry2):
                    def edge_body(i, carry3):
                        e = g * 16 + i
                        a = (za_v[b, e, pl.ds(0, 16)]
                             * zb_v[b, e, pl.ds(0, 16)])
                        for j in range(1, D // 16):
                            a = a + (za_v[b, e, pl.ds(j * 16, 16)]
                                     * zb_v[b, e, pl.ds(j * 16, 16)])
                        p_v[i, pl.ds(0, 16)] = a
                        return carry3

                    lax.fori_loop(0, 16, edge_body, 0)
                    acc = plsc.load_gather(
                        p_v, [lane, jnp.zeros((16,), jnp.int32)])
                    for i in range(1, 16):
                        acc = acc + plsc.load_gather(
                            p_v, [lane, jnp.full((16,), i, jnp.int32)])
                    sc_v[b, pl.ds(g * 16, 16)] = acc
                    return carry2

                lax.fori_loop(0, CHUNK // 16, group_body, 0)
                pltpu.async_copy(
                    sc_v.at[b],
                    out_hbm.at[pl.ds((base + c) * CHUNK, CHUNK)], semw[b])

                @pl.when(c + 4 < count)
                def _():
                    pltpu.make_async_copy(
                        pk_hbm.at[base + c + 4], pk_v.at[b], semi[b]).wait()
                    _copy_idx(pk_v, sidx_v, didx_v, b)
                    pltpu.async_copy(
                        z_hbm.at[sidx_v.at[b]], za_v.at[b], semd[b])
                    pltpu.async_copy(
                        z_hbm.at[didx_v.at[b]], zb_v.at[b], semd[b])

                @pl.when(c + 8 < count)
                def _():
                    pltpu.async_copy(pk_hbm.at[base + c + 8], pk_v.at[b],
                                     semi[b])
            return carry

        lax.fori_loop(0, count // 4, quad_body, 0)
        for b in range(4):
            pltpu.make_async_copy(
                sc_v.at[b], out_hbm.at[pl.ds(0, CHUNK)], semw[b]).wait()


def _dec_pass(z, pkr):
    return pl.kernel(
        _dec_body,
        out_type=jax.ShapeDtypeStruct((EPAD,), jnp.float32),
        mesh=plsc.VectorSubcoreMesh(core_axis_name="c", subcore_axis_name="s"),
        compiler_params=pltpu.CompilerParams(needs_layout_passes=False),
        scratch_types=[
            pltpu.VMEM((4, 2, CHUNK), jnp.int32),
            pltpu.VMEM((4, CHUNK), jnp.int32),
            pltpu.VMEM((4, CHUNK), jnp.int32),
            pltpu.VMEM((4, CHUNK, D), jnp.float32),
            pltpu.VMEM((4, CHUNK, D), jnp.float32),
            pltpu.VMEM((16, 17), jnp.float32),
            pltpu.VMEM((4, CHUNK), jnp.float32),
        ] + [pltpu.SemaphoreType.DMA] * 12,
    )(z, pkr)


def _dis_block(dp_block, block_id):
    deg = jnp.sum(dp_block, axis=0)
    rows = block_id * BLK + lax.broadcasted_iota(jnp.int32, (BLK,), 0)
    real = rows < N
    deg = deg + real.astype(jnp.float32)
    dis = jnp.where(real, lax.rsqrt(deg), 0.0)
    return dis[:, None]


def _tc1_body(x_ref, w_ref, dp_ref, o_ref):
    i = pl.program_id(0)
    dis = _dis_block(dp_ref[...], i)
    xw = jnp.dot(x_ref[...], w_ref[...], preferred_element_type=jnp.float32)
    o_ref[...] = xw * dis


def _tc1(xpad, W1, deg_part):
    return pl.pallas_call(
        _tc1_body,
        out_shape=jax.ShapeDtypeStruct((NPAD, D), jnp.float32),
        grid=(NBLK,),
        in_specs=[
            pl.BlockSpec((BLK, D), lambda i: (i, 0)),
            pl.BlockSpec((D, D), lambda i: (0, 0)),
            pl.BlockSpec((NW, BLK), lambda i: (0, i)),
        ],
        out_specs=pl.BlockSpec((BLK, D), lambda i: (i, 0)),
    )(xpad, W1, deg_part)


def _tc2_body(acc_ref, xws_ref, w_ref, b_ref, dp_ref, o_ref):
    i = pl.program_id(0)
    dis = _dis_block(dp_ref[...], i)
    s = (acc_ref[0] + acc_ref[1] + xws_ref[...]) * dis
    h = jnp.maximum(s + b_ref[...], 0.0)
    o_ref[...] = jnp.dot(h, w_ref[...], preferred_element_type=jnp.float32) * dis


def _tc2(acc, xws1, W2, b1r, deg_part):
    return pl.pallas_call(
        _tc2_body,
        out_shape=jax.ShapeDtypeStruct((NPAD, D), jnp.float32),
        grid=(NBLK,),
        in_specs=[
            pl.BlockSpec((NC, BLK, D), lambda i: (0, i, 0)),
            pl.BlockSpec((BLK, D), lambda i: (i, 0)),
            pl.BlockSpec((D, D), lambda i: (0, 0)),
            pl.BlockSpec((1, D), lambda i: (0, 0)),
            pl.BlockSpec((NW, BLK), lambda i: (0, i)),
        ],
        out_specs=pl.BlockSpec((BLK, D), lambda i: (i, 0)),
    )(acc, xws1, W2, b1r, deg_part)


def _tc3_body(acc_ref, xws_ref, b_ref, dp_ref, o_ref):
    i = pl.program_id(0)
    dis = _dis_block(dp_ref[...], i)
    o_ref[...] = (acc_ref[0] + acc_ref[1] + xws_ref[...]) * dis + b_ref[...]


def _tc3(acc, xws2, b2r, deg_part):
    return pl.pallas_call(
        _tc3_body,
        out_shape=jax.ShapeDtypeStruct((NPAD, D), jnp.float32),
        grid=(NBLK,),
        in_specs=[
            pl.BlockSpec((NC, BLK, D), lambda i: (0, i, 0)),
            pl.BlockSpec((BLK, D), lambda i: (i, 0)),
            pl.BlockSpec((1, D), lambda i: (0, 0)),
            pl.BlockSpec((NW, BLK), lambda i: (0, i)),
        ],
        out_specs=pl.BlockSpec((BLK, D), lambda i: (i, 0)),
    )(acc, xws2, b2r, deg_part)


def kernel(x, edge_index, W1, b1, W2, b2):
    ei = edge_index.astype(jnp.int32)
    src = ei[0]
    dst = ei[1]
    pad = jnp.full((EPAD - E,), N, jnp.int32)
    srcp = jnp.concatenate([src, pad])
    dstp = jnp.concatenate([dst, pad])
    pkr = jnp.stack(
        [srcp.reshape(NCHUNKS, CHUNK), dstp.reshape(NCHUNKS, CHUNK)], axis=1)
    dst16 = dstp.reshape(NW, DEG_CH, 16)

    xpad = jnp.concatenate(
        [x.astype(jnp.float32), jnp.zeros((NPAD - N, D), jnp.float32)])
    zeros_tab = jnp.zeros((NPAD, D), jnp.float32)
    b1r = b1.reshape(1, D).astype(jnp.float32)
    b2r = b2.reshape(1, D).astype(jnp.float32)

    deg_part = _deg_pass(dst16)
    xws1 = _tc1(xpad, W1, deg_part)
    acc1 = _spmm_pass(xws1, pkr, zeros_tab)
    xws2 = _tc2(acc1, xws1, W2, b1r, deg_part)
    acc2 = _spmm_pass(xws2, pkr, zeros_tab)
    z = _tc3(acc2, xws2, b2r, deg_part)
    scores_pad = _dec_pass(z, pkr)
    return scores_pad[:E]


# speedup vs baseline: 1.3280x; 1.0066x over previous
import jax
import jax.numpy as jnp
from jax import lax
from jax.experimental import pallas as pl
from jax.experimental.pallas import tpu as pltpu
from jax.experimental.pallas import tpu_sc as plsc

N = 10000
D = 128
E = 320000

NC = 2
NS = 16
NW = NC * NS

NPAD = 10240
EPW = 10240
EPAD = NW * EPW
DEG_CH = EPW // 16
RPT = NPAD // NS

CHUNK = 64
NCHUNKS = EPAD // CHUNK
SPT = NCHUNKS // NS
SN0 = 256
SN1 = 64
DN0 = 304
DN1 = 16

BLK = 512
NBLK = NPAD // BLK


def _deg_body(didx_hbm, out_hbm, didx_v, deg_v):
    cid = lax.axis_index("c")
    sid = lax.axis_index("s")
    wid = cid * NS + sid
    pltpu.sync_copy(didx_hbm.at[wid], didx_v)

    def zero_body(i, carry):
        deg_v[pl.ds(i * 16, 16)] = jnp.zeros((16,), jnp.float32)
        return carry

    lax.fori_loop(0, NPAD // 16, zero_body, 0)

    ones = jnp.ones((16,), jnp.float32)

    def acc_body(c, carry):
        idx = didx_v[c]
        plsc.addupdate_scatter(deg_v, [idx], ones)
        return carry

    lax.fori_loop(0, DEG_CH, acc_body, 0)
    pltpu.sync_copy(deg_v, out_hbm.at[wid])


def _deg_pass(dst16):
    return pl.kernel(
        _deg_body,
        out_type=jax.ShapeDtypeStruct((NW, NPAD), jnp.float32),
        mesh=plsc.VectorSubcoreMesh(core_axis_name="c", subcore_axis_name="s"),
        compiler_params=pltpu.CompilerParams(needs_layout_passes=False),
        scratch_types=[
            pltpu.VMEM((DEG_CH, 16), jnp.int32),
            pltpu.VMEM((NPAD,), jnp.float32),
        ],
    )(dst16)


def _copy_idx(pk_v, sidx_v, didx_v, slot):
    for w in range(CHUNK // 16):
        sidx_v[slot, pl.ds(w * 16, 16)] = pk_v[slot, 0, pl.ds(w * 16, 16)]
        didx_v[slot, pl.ds(w * 16, 16)] = pk_v[slot, 1, pl.ds(w * 16, 16)]


def _spmm_body(table_hbm, pk_hbm, zeros_hbm, out_hbm,
               pk_v, sidx_v, didx_v, rows_v, acc_sh, *sems):
    semr = sems[0:4]
    semw = sems[4:8]
    semi = sems[8:12]
    cid = lax.axis_index("c")
    sid = lax.axis_index("s")
    r0 = sid * RPT

    if True:
        base = jnp.where(cid == 0, sid * SN0, NS * SN0 + sid * SN1)
        count = jnp.where(cid == 0, SN0, SN1)
        pltpu.sync_copy(zeros_hbm.at[pl.ds(r0, RPT)],
                        acc_sh.at[pl.ds(r0, RPT)])
        plsc.subcore_barrier()

        for k in range(2):
            pltpu.sync_copy(pk_hbm.at[base + k], pk_v.at[k])
            _copy_idx(pk_v, sidx_v, didx_v, k)
            pltpu.async_copy(
                table_hbm.at[sidx_v.at[k]], rows_v.at[k], semr[k])
        for j in range(2, 6):
            pltpu.async_copy(pk_hbm.at[base + j], pk_v.at[j % 4],
                             semi[j % 4])

        def quad_body(c4, carry):
            for k in range(4):
                c = 4 * c4 + k
                b = k
                b2 = (k + 2) % 4
                pltpu.make_async_copy(
                    table_hbm.at[sidx_v.at[b]], rows_v.at[b],
                    semr[b]).wait()
                pltpu.async_copy(
                    rows_v.at[b], acc_sh.at[didx_v.at[b]], semw[b],
                    add=True)

                def _drain():
                    pltpu.make_async_copy(
                        rows_v.at[b2], acc_sh.at[didx_v.at[b2]],
                        semw[b2]).wait()

                if k >= 2:
                    _drain()
                else:
                    @pl.when(c4 > 0)
                    def _():
                        _drain()

                @pl.when(c + 2 < count)
                def _():
                    pltpu.make_async_copy(
                        pk_hbm.at[base + c + 2], pk_v.at[b2],
                        semi[b2]).wait()
                    _copy_idx(pk_v, sidx_v, didx_v, b2)
                    pltpu.async_copy(
                        table_hbm.at[sidx_v.at[b2]], rows_v.at[b2],
                        semr[b2])

                @pl.when(c + 6 < count)
                def _():
                    pltpu.async_copy(pk_hbm.at[base + c + 6], pk_v.at[b2],
                                     semi[b2])
            return carry

        lax.fori_loop(0, count // 4, quad_body, 0)
        for b in (2, 3):
            pltpu.make_async_copy(
                rows_v.at[b], acc_sh.at[didx_v.at[b]], semw[b]).wait()
        plsc.subcore_barrier()
        pltpu.sync_copy(acc_sh.at[pl.ds(r0, RPT)],
                        out_hbm.at[cid, pl.ds(r0, RPT)])


def _spmm_pass(table, pkr, zeros_tab):
    return pl.kernel(
        _spmm_body,
        out_type=jax.ShapeDtypeStruct((NC, NPAD, D), jnp.float32),
        mesh=plsc.VectorSubcoreMesh(core_axis_name="c", subcore_axis_name="s"),
        compiler_params=pltpu.CompilerParams(needs_layout_passes=False),
        scratch_types=[
            pltpu.VMEM((4, 2, CHUNK), jnp.int32),
            pltpu.VMEM((4, CHUNK), jnp.int32),
            pltpu.VMEM((4, CHUNK), jnp.int32),
            pltpu.VMEM((4, CHUNK, D), jnp.float32),
            pltpu.VMEM_SHARED((NPAD, D), jnp.float32),
        ] + [pltpu.SemaphoreType.DMA] * 12,
    )(table, pkr, zeros_tab)


def _dec_body(z_hbm, pk_hbm, out_hbm,
              pk_v, sidx_v, didx_v, za_v, zb_v, p_v, sc_v, *sems):
    semd = sems[0:4]
    semw = sems[4:8]
    semi = sems[8:12]
    cid = lax.axis_index("c")
    sid = lax.axis_index("s")
    lane = lax.broadcasted_iota(jnp.int32, (16,), 0)

    if True:
        base = jnp.where(cid == 0, sid * DN0, NS * DN0 + sid * DN1)
        count = jnp.where(cid == 0, DN0, DN1)

        for k in range(4):
            pltpu.sync_copy(pk_hbm.at[base + k], pk_v.at[k])
            _copy_idx(pk_v, sidx_v, didx_v, k)
            pltpu.async_copy(z_hbm.at[sidx_v.at[k]], za_v.at[k], semd[k])
            pltpu.async_copy(z_hbm.at[didx_v.at[k]], zb_v.at[k], semd[k])
        for j in range(4, 8):
            pltpu.async_copy(pk_hbm.at[base + j], pk_v.at[j % 4],
                             semi[j % 4])

        def quad_body(c4, carry):
            for k in range(4):
                c = 4 * c4 + k
                b = k
                pltpu.make_async_copy(
                    z_hbm.at[sidx_v.at[b]], za_v.at[b], semd[b]).wait()
                pltpu.make_async_copy(
                    z_hbm.at[didx_v.at[b]], zb_v.at[b], semd[b]).wait()

                @pl.when(c4 > 0)
                def _():
                    pltpu.make_async_copy(
                        sc_v.at[b], out_hbm.at[pl.ds(0, CHUNK)],
                        semw[b]).wait()

                def group_body(g, carry2):
                    def edge_body(i, carry3):
                        e = g * 16 + i
                        a = (za_v[b, e, pl.ds(0, 16)]
                             * zb_v[b, e, pl.ds(0, 16)])
                        for j in range(1, D // 16):
                            a = a + (za_v[b, e, pl.ds(j * 16, 16)]
                                     * zb_v[b, e, pl.ds(j * 16, 16)])
                        p_v[i, pl.ds(0, 16)] = a
                        return carry3

                    lax.fori_loop(0, 16, edge_body, 0)
                    acc = plsc.load_gather(
                        p_v, [lane, jnp.zeros((16,), jnp.int32)])
                    for i in range(1, 16):
                        acc = acc + plsc.load_gather(
                            p_v, [lane, jnp.full((16,), i, jnp.int32)])
                    sc_v[b, pl.ds(g * 16, 16)] = acc
                    return carry2

                lax.fori_loop(0, CHUNK // 16, group_body, 0)
                pltpu.async_copy(
                    sc_v.at[b],
                    out_hbm.at[pl.ds((base + c) * CHUNK, CHUNK)], semw[b])

                @pl.when(c + 4 < count)
                def _():
                    pltpu.make_async_copy(
                        pk_hbm.at[base + c + 4], pk_v.at[b], semi[b]).wait()
                    _copy_idx(pk_v, sidx_v, didx_v, b)
                    pltpu.async_copy(
                        z_hbm.at[sidx_v.at[b]], za_v.at[b], semd[b])
                    pltpu.async_copy(
                        z_hbm.at[didx_v.at[b]], zb_v.at[b], semd[b])

                @pl.when(c + 8 < count)
                def _():
                    pltpu.async_copy(pk_hbm.at[base + c + 8], pk_v.at[b],
                                     semi[b])
            return carry

        lax.fori_loop(0, count // 4, quad_body, 0)
        for b in range(4):
            pltpu.make_async_copy(
                sc_v.at[b], out_hbm.at[pl.ds(0, CHUNK)], semw[b]).wait()


def _dec_pass(z, pkr):
    return pl.kernel(
        _dec_body,
        out_type=jax.ShapeDtypeStruct((EPAD,), jnp.float32),
        mesh=plsc.VectorSubcoreMesh(core_axis_name="c", subcore_axis_name="s"),
        compiler_params=pltpu.CompilerParams(needs_layout_passes=False),
        scratch_types=[
            pltpu.VMEM((4, 2, CHUNK), jnp.int32),
            pltpu.VMEM((4, CHUNK), jnp.int32),
            pltpu.VMEM((4, CHUNK), jnp.int32),
            pltpu.VMEM((4, CHUNK, D), jnp.float32),
            pltpu.VMEM((4, CHUNK, D), jnp.float32),
            pltpu.VMEM((16, 17), jnp.float32),
            pltpu.VMEM((4, CHUNK), jnp.float32),
        ] + [pltpu.SemaphoreType.DMA] * 12,
    )(z, pkr)


def _dis_block(dp_block, block_id):
    deg = jnp.sum(dp_block, axis=0)
    rows = block_id * BLK + lax.broadcasted_iota(jnp.int32, (BLK,), 0)
    real = rows < N
    deg = deg + real.astype(jnp.float32)
    dis = jnp.where(real, lax.rsqrt(deg), 0.0)
    return dis[:, None]


def _tc1_body(x_ref, w_ref, dp_ref, o_ref):
    i = pl.program_id(0)
    dis = _dis_block(dp_ref[...], i)
    xw = jnp.dot(x_ref[...], w_ref[...], preferred_element_type=jnp.float32)
    o_ref[...] = xw * dis


def _tc1(xpad, W1, deg_part):
    return pl.pallas_call(
        _tc1_body,
        out_shape=jax.ShapeDtypeStruct((NPAD, D), jnp.float32),
        grid=(NBLK,),
        in_specs=[
            pl.BlockSpec((BLK, D), lambda i: (i, 0)),
            pl.BlockSpec((D, D), lambda i: (0, 0)),
            pl.BlockSpec((NW, BLK), lambda i: (0, i)),
        ],
        out_specs=pl.BlockSpec((BLK, D), lambda i: (i, 0)),
    )(xpad, W1, deg_part)


def _tc2_body(acc_ref, xws_ref, w_ref, b_ref, dp_ref, o_ref):
    i = pl.program_id(0)
    dis = _dis_block(dp_ref[...], i)
    s = (acc_ref[0] + acc_ref[1] + xws_ref[...]) * dis
    h = jnp.maximum(s + b_ref[...], 0.0)
    o_ref[...] = jnp.dot(h, w_ref[...], preferred_element_type=jnp.float32) * dis


def _tc2(acc, xws1, W2, b1r, deg_part):
    return pl.pallas_call(
        _tc2_body,
        out_shape=jax.ShapeDtypeStruct((NPAD, D), jnp.float32),
        grid=(NBLK,),
        in_specs=[
            pl.BlockSpec((NC, BLK, D), lambda i: (0, i, 0)),
            pl.BlockSpec((BLK, D), lambda i: (i, 0)),
            pl.BlockSpec((D, D), lambda i: (0, 0)),
            pl.BlockSpec((1, D), lambda i: (0, 0)),
            pl.BlockSpec((NW, BLK), lambda i: (0, i)),
        ],
        out_specs=pl.BlockSpec((BLK, D), lambda i: (i, 0)),
    )(acc, xws1, W2, b1r, deg_part)


def _tc3_body(acc_ref, xws_ref, b_ref, dp_ref, o_ref):
    i = pl.program_id(0)
    dis = _dis_block(dp_ref[...], i)
    o_ref[...] = (acc_ref[0] + acc_ref[1] + xws_ref[...]) * dis + b_ref[...]


def _tc3(acc, xws2, b2r, deg_part):
    return pl.pallas_call(
        _tc3_body,
        out_shape=jax.ShapeDtypeStruct((NPAD, D), jnp.float32),
        grid=(NBLK,),
        in_specs=[
            pl.BlockSpec((NC, BLK, D), lambda i: (0, i, 0)),
            pl.BlockSpec((BLK, D), lambda i: (i, 0)),
            pl.BlockSpec((1, D), lambda i: (0, 0)),
            pl.BlockSpec((NW, BLK), lambda i: (0, i)),
        ],
        out_specs=pl.BlockSpec((BLK, D), lambda i: (i, 0)),
    )(acc, xws2, b2r, deg_part)


def kernel(x, edge_index, W1, b1, W2, b2):
    ei = edge_index.astype(jnp.int32)
    src = ei[0]
    dst = ei[1]
    pad = jnp.full((EPAD - E,), N, jnp.int32)
    srcp = jnp.concatenate([src, pad])
    dstp = jnp.concatenate([dst, pad])
    pkr = jnp.stack(
        [srcp.reshape(NCHUNKS, CHUNK), dstp.reshape(NCHUNKS, CHUNK)], axis=1)
    dst16 = dstp.reshape(NW, DEG_CH, 16)

    xpad = jnp.concatenate(
        [x.astype(jnp.float32), jnp.zeros((NPAD - N, D), jnp.float32)])
    zeros_tab = jnp.zeros((NPAD, D), jnp.float32)
    b1r = b1.reshape(1, D).astype(jnp.float32)
    b2r = b2.reshape(1, D).astype(jnp.float32)

    deg_part = _deg_pass(dst16)
    xws1 = _tc1(xpad, W1, deg_part)
    acc1 = _spmm_pass(xws1, pkr, zeros_tab)
    xws2 = _tc2(acc1, xws1, W2, b1r, deg_part)
    acc2 = _spmm_pass(xws2, pkr, zeros_tab)
    z = _tc3(acc2, xws2, b2r, deg_part)
    scores_pad = _dec_pass(z, pkr)
    return scores_pad[:E]


# speedup vs baseline: 1.3663x; 1.0288x over previous
import jax
import jax.numpy as jnp
from jax import lax
from jax.experimental import pallas as pl
from jax.experimental.pallas import tpu as pltpu
from jax.experimental.pallas import tpu_sc as plsc

N = 10000
D = 128
E = 320000

NC = 2
NS = 16
NW = NC * NS

NPAD = 10240
EPW = 10240
EPAD = NW * EPW
DEG_CH = EPW // 16
RPT = NPAD // NS

CHUNK = 64
NCHUNKS = EPAD // CHUNK
SPT = NCHUNKS // NS
SN0 = 272
SN1 = 48
DN0 = 304
DN1 = 16

BLK = 512
NBLK = NPAD // BLK


def _deg_body(didx_hbm, out_hbm, didx_v, deg_v):
    cid = lax.axis_index("c")
    sid = lax.axis_index("s")
    wid = cid * NS + sid
    pltpu.sync_copy(didx_hbm.at[wid], didx_v)

    def zero_body(i, carry):
        deg_v[pl.ds(i * 16, 16)] = jnp.zeros((16,), jnp.float32)
        return carry

    lax.fori_loop(0, NPAD // 16, zero_body, 0)

    ones = jnp.ones((16,), jnp.float32)

    def acc_body(c, carry):
        idx = didx_v[c]
        plsc.addupdate_scatter(deg_v, [idx], ones)
        return carry

    lax.fori_loop(0, DEG_CH, acc_body, 0)
    pltpu.sync_copy(deg_v, out_hbm.at[wid])


def _deg_pass(dst16):
    return pl.kernel(
        _deg_body,
        out_type=jax.ShapeDtypeStruct((NW, NPAD), jnp.float32),
        mesh=plsc.VectorSubcoreMesh(core_axis_name="c", subcore_axis_name="s"),
        compiler_params=pltpu.CompilerParams(needs_layout_passes=False),
        scratch_types=[
            pltpu.VMEM((DEG_CH, 16), jnp.int32),
            pltpu.VMEM((NPAD,), jnp.float32),
        ],
    )(dst16)


def _copy_idx(pk_v, sidx_v, didx_v, slot):
    for w in range(CHUNK // 16):
        sidx_v[slot, pl.ds(w * 16, 16)] = pk_v[slot, 0, pl.ds(w * 16, 16)]
        didx_v[slot, pl.ds(w * 16, 16)] = pk_v[slot, 1, pl.ds(w * 16, 16)]


def _spmm_body(table_hbm, pk_hbm, zeros_hbm, out_hbm,
               pk_v, sidx_v, didx_v, rows_v, acc_sh, *sems):
    semr = sems[0:4]
    semw = sems[4:8]
    semi = sems[8:12]
    cid = lax.axis_index("c")
    sid = lax.axis_index("s")
    r0 = sid * RPT

    if True:
        base = jnp.where(cid == 0, sid * SN0, NS * SN0 + sid * SN1)
        count = jnp.where(cid == 0, SN0, SN1)
        pltpu.sync_copy(zeros_hbm.at[pl.ds(r0, RPT)],
                        acc_sh.at[pl.ds(r0, RPT)])
        plsc.subcore_barrier()

        for k in range(2):
            pltpu.sync_copy(pk_hbm.at[base + k], pk_v.at[k])
            _copy_idx(pk_v, sidx_v, didx_v, k)
            pltpu.async_copy(
                table_hbm.at[sidx_v.at[k]], rows_v.at[k], semr[k])
        for j in range(2, 6):
            pltpu.async_copy(pk_hbm.at[base + j], pk_v.at[j % 4],
                             semi[j % 4])

        def quad_body(c4, carry):
            for k in range(4):
                c = 4 * c4 + k
                b = k
                b2 = (k + 2) % 4
                pltpu.make_async_copy(
                    table_hbm.at[sidx_v.at[b]], rows_v.at[b],
                    semr[b]).wait()
                pltpu.async_copy(
                    rows_v.at[b], acc_sh.at[didx_v.at[b]], semw[b],
                    add=True)

                def _drain():
                    pltpu.make_async_copy(
                        rows_v.at[b2], acc_sh.at[didx_v.at[b2]],
                        semw[b2]).wait()

                if k >= 2:
                    _drain()
                else:
                    @pl.when(c4 > 0)
                    def _():
                        _drain()

                @pl.when(c + 2 < count)
                def _():
                    pltpu.make_async_copy(
                        pk_hbm.at[base + c + 2], pk_v.at[b2],
                        semi[b2]).wait()
                    _copy_idx(pk_v, sidx_v, didx_v, b2)
                    pltpu.async_copy(
                        table_hbm.at[sidx_v.at[b2]], rows_v.at[b2],
                        semr[b2])

                @pl.when(c + 6 < count)
                def _():
                    pltpu.async_copy(pk_hbm.at[base + c + 6], pk_v.at[b2],
                                     semi[b2])
            return carry

        lax.fori_loop(0, count // 4, quad_body, 0)
        for b in (2, 3):
            pltpu.make_async_copy(
                rows_v.at[b], acc_sh.at[didx_v.at[b]], semw[b]).wait()
        plsc.subcore_barrier()
        pltpu.sync_copy(acc_sh.at[pl.ds(r0, RPT)],
                        out_hbm.at[cid, pl.ds(r0, RPT)])


def _spmm_pass(table, pkr, zeros_tab):
    return pl.kernel(
        _spmm_body,
        out_type=jax.ShapeDtypeStruct((NC, NPAD, D), jnp.float32),
        mesh=plsc.VectorSubcoreMesh(core_axis_name="c", subcore_axis_name="s"),
        compiler_params=pltpu.CompilerParams(needs_layout_passes=False),
        scratch_types=[
            pltpu.VMEM((4, 2, CHUNK), jnp.int32),
            pltpu.VMEM((4, CHUNK), jnp.int32),
            pltpu.VMEM((4, CHUNK), jnp.int32),
            pltpu.VMEM((4, CHUNK, D), jnp.float32),
            pltpu.VMEM_SHARED((NPAD, D), jnp.float32),
        ] + [pltpu.SemaphoreType.DMA] * 12,
    )(table, pkr, zeros_tab)


def _dec_body(z_hbm, pk_hbm, out_hbm,
              pk_v, sidx_v, didx_v, za_v, zb_v, p_v, sc_v, *sems):
    semd = sems[0:4]
    semw = sems[4:8]
    semi = sems[8:12]
    cid = lax.axis_index("c")
    sid = lax.axis_index("s")
    lane = lax.broadcasted_iota(jnp.int32, (16,), 0)

    if True:
        base = jnp.where(cid == 0, sid * DN0, NS * DN0 + sid * DN1)
        count = jnp.where(cid == 0, DN0, DN1)

        for k in range(4):
            pltpu.sync_copy(pk_hbm.at[base + k], pk_v.at[k])
            _copy_idx(pk_v, sidx_v, didx_v, k)
            pltpu.async_copy(z_hbm.at[sidx_v.at[k]], za_v.at[k], semd[k])
            pltpu.async_copy(z_hbm.at[didx_v.at[k]], zb_v.at[k], semd[k])
        for j in range(4, 8):
            pltpu.async_copy(pk_hbm.at[base + j], pk_v.at[j % 4],
                             semi[j % 4])

        def quad_body(c4, carry):
            for k in range(4):
                c = 4 * c4 + k
                b = k
                pltpu.make_async_copy(
                    z_hbm.at[sidx_v.at[b]], za_v.at[b], semd[b]).wait()
                pltpu.make_async_copy(
                    z_hbm.at[didx_v.at[b]], zb_v.at[b], semd[b]).wait()

                @pl.when(c4 > 0)
                def _():
                    pltpu.make_async_copy(
                        sc_v.at[b], out_hbm.at[pl.ds(0, CHUNK)],
                        semw[b]).wait()

                def group_body(g, carry2):
                    def edge_body(i, carry3):
                        e = g * 16 + i
                        a = (za_v[b, e, pl.ds(0, 16)]
                             * zb_v[b, e, pl.ds(0, 16)])
                        for j in range(1, D // 16):
                            a = a + (za_v[b, e, pl.ds(j * 16, 16)]
                                     * zb_v[b, e, pl.ds(j * 16, 16)])
                        p_v[i, pl.ds(0, 16)] = a
                        return carry3

                    lax.fori_loop(0, 16, edge_body, 0)
                    acc = plsc.load_gather(
                        p_v, [lane, jnp.zeros((16,), jnp.int32)])
                    for i in range(1, 16):
                        acc = acc + plsc.load_gather(
                            p_v, [lane, jnp.full((16,), i, jnp.int32)])
                    sc_v[b, pl.ds(g * 16, 16)] = acc
                    return carry2

                lax.fori_loop(0, CHUNK // 16, group_body, 0)
                pltpu.async_copy(
                    sc_v.at[b],
                    out_hbm.at[pl.ds((base + c) * CHUNK, CHUNK)], semw[b])

                @pl.when(c + 4 < count)
                def _():
                    pltpu.make_async_copy(
                        pk_hbm.at[base + c + 4], pk_v.at[b], semi[b]).wait()
                    _copy_idx(pk_v, sidx_v, didx_v, b)
                    pltpu.async_copy(
                        z_hbm.at[sidx_v.at[b]], za_v.at[b], semd[b])
                    pltpu.async_copy(
                        z_hbm.at[didx_v.at[b]], zb_v.at[b], semd[b])

                @pl.when(c + 8 < count)
                def _():
                    pltpu.async_copy(pk_hbm.at[base + c + 8], pk_v.at[b],
                                     semi[b])
            return carry

        lax.fori_loop(0, count // 4, quad_body, 0)
        for b in range(4):
            pltpu.make_async_copy(
                sc_v.at[b], out_hbm.at[pl.ds(0, CHUNK)], semw[b]).wait()


def _dec_pass(z, pkr):
    return pl.kernel(
        _dec_body,
        out_type=jax.ShapeDtypeStruct((EPAD,), jnp.float32),
        mesh=plsc.VectorSubcoreMesh(core_axis_name="c", subcore_axis_name="s"),
        compiler_params=pltpu.CompilerParams(needs_layout_passes=False),
        scratch_types=[
            pltpu.VMEM((4, 2, CHUNK), jnp.int32),
            pltpu.VMEM((4, CHUNK), jnp.int32),
            pltpu.VMEM((4, CHUNK), jnp.int32),
            pltpu.VMEM((4, CHUNK, D), jnp.float32),
            pltpu.VMEM((4, CHUNK, D), jnp.float32),
            pltpu.VMEM((16, 17), jnp.float32),
            pltpu.VMEM((4, CHUNK), jnp.float32),
        ] + [pltpu.SemaphoreType.DMA] * 12,
    )(z, pkr)


def _dis_block(dp_block, block_id):
    deg = jnp.sum(dp_block, axis=0)
    rows = block_id * BLK + lax.broadcasted_iota(jnp.int32, (BLK,), 0)
    real = rows < N
    deg = deg + real.astype(jnp.float32)
    dis = jnp.where(real, lax.rsqrt(deg), 0.0)
    return dis[:, None]


def _tc1_body(x_ref, w_ref, dp_ref, o_ref):
    i = pl.program_id(0)
    dis = _dis_block(dp_ref[...], i)
    xw = jnp.dot(x_ref[...], w_ref[...], preferred_element_type=jnp.float32)
    o_ref[...] = xw * dis


def _tc1(xpad, W1, deg_part):
    return pl.pallas_call(
        _tc1_body,
        out_shape=jax.ShapeDtypeStruct((NPAD, D), jnp.float32),
        grid=(NBLK,),
        in_specs=[
            pl.BlockSpec((BLK, D), lambda i: (i, 0)),
            pl.BlockSpec((D, D), lambda i: (0, 0)),
            pl.BlockSpec((NW, BLK), lambda i: (0, i)),
        ],
        out_specs=pl.BlockSpec((BLK, D), lambda i: (i, 0)),
    )(xpad, W1, deg_part)


def _tc2_body(acc_ref, xws_ref, w_ref, b_ref, dp_ref, o_ref):
    i = pl.program_id(0)
    dis = _dis_block(dp_ref[...], i)
    s = (acc_ref[0] + acc_ref[1] + xws_ref[...]) * dis
    h = jnp.maximum(s + b_ref[...], 0.0)
    o_ref[...] = jnp.dot(h, w_ref[...], preferred_element_type=jnp.float32) * dis


def _tc2(acc, xws1, W2, b1r, deg_part):
    return pl.pallas_call(
        _tc2_body,
        out_shape=jax.ShapeDtypeStruct((NPAD, D), jnp.float32),
        grid=(NBLK,),
        in_specs=[
            pl.BlockSpec((NC, BLK, D), lambda i: (0, i, 0)),
            pl.BlockSpec((BLK, D), lambda i: (i, 0)),
            pl.BlockSpec((D, D), lambda i: (0, 0)),
            pl.BlockSpec((1, D), lambda i: (0, 0)),
            pl.BlockSpec((NW, BLK), lambda i: (0, i)),
        ],
        out_specs=pl.BlockSpec((BLK, D), lambda i: (i, 0)),
    )(acc, xws1, W2, b1r, deg_part)


def _tc3_body(acc_ref, xws_ref, b_ref, dp_ref, o_ref):
    i = pl.program_id(0)
    dis = _dis_block(dp_ref[...], i)
    o_ref[...] = (acc_ref[0] + acc_ref[1] + xws_ref[...]) * dis + b_ref[...]


def _tc3(acc, xws2, b2r, deg_part):
    return pl.pallas_call(
        _tc3_body,
        out_shape=jax.ShapeDtypeStruct((NPAD, D), jnp.float32),
        grid=(NBLK,),
        in_specs=[
            pl.BlockSpec((NC, BLK, D), lambda i: (0, i, 0)),
            pl.BlockSpec((BLK, D), lambda i: (i, 0)),
            pl.BlockSpec((1, D), lambda i: (0, 0)),
            pl.BlockSpec((NW, BLK), lambda i: (0, i)),
        ],
        out_specs=pl.BlockSpec((BLK, D), lambda i: (i, 0)),
    )(acc, xws2, b2r, deg_part)


def kernel(x, edge_index, W1, b1, W2, b2):
    ei = edge_index.astype(jnp.int32)
    src = ei[0]
    dst = ei[1]
    pad = jnp.full((EPAD - E,), N, jnp.int32)
    srcp = jnp.concatenate([src, pad])
    dstp = jnp.concatenate([dst, pad])
    pkr = jnp.stack(
        [srcp.reshape(NCHUNKS, CHUNK), dstp.reshape(NCHUNKS, CHUNK)], axis=1)
    dst16 = dstp.reshape(NW, DEG_CH, 16)

    xpad = jnp.concatenate(
        [x.astype(jnp.float32), jnp.zeros((NPAD - N, D), jnp.float32)])
    zeros_tab = jnp.zeros((NPAD, D), jnp.float32)
    b1r = b1.reshape(1, D).astype(jnp.float32)
    b2r = b2.reshape(1, D).astype(jnp.float32)

    deg_part = _deg_pass(dst16)
    xws1 = _tc1(xpad, W1, deg_part)
    acc1 = _spmm_pass(xws1, pkr, zeros_tab)
    xws2 = _tc2(acc1, xws1, W2, b1r, deg_part)
    acc2 = _spmm_pass(xws2, pkr, zeros_tab)
    z = _tc3(acc2, xws2, b2r, deg_part)
    scores_pad = _dec_pass(z, pkr)
    return scores_pad[:E]
